# baseline (device time: 110228 ns/iter reference)
import jax
import jax.numpy as jnp
from jax import lax
from jax.experimental import pallas as pl
from jax.experimental.pallas import tpu as pltpu

N_DEV = 4
E = 32
E_LOCAL = 8
CAP = 51
SLOTS = 64
M = E_LOCAL * SLOTS


def _allgather_body(yg_ref, out_ref, send_sems, recv_sems):
    my = lax.axis_index("i")

    barrier_sem = pltpu.get_barrier_semaphore()
    for off in (1, 2, 3):
        pl.semaphore_signal(
            barrier_sem, inc=1,
            device_id=((my + off) % N_DEV,),
            device_id_type=pl.DeviceIdType.MESH,
        )
    pl.semaphore_wait(barrier_sem, 3)

    out_ref[pl.ds(my * M, M), :] = yg_ref[...]

    sends = []
    for off in (1, 2, 3):
        rdma = pltpu.make_async_remote_copy(
            src_ref=out_ref.at[pl.ds(my * M, M), :],
            dst_ref=out_ref.at[pl.ds(my * M, M), :],
            send_sem=send_sems.at[off - 1],
            recv_sem=recv_sems.at[off - 1],
            device_id=((my + off) % N_DEV,),
            device_id_type=pl.DeviceIdType.MESH,
        )
        rdma.start()
        sends.append(rdma)

    for off in (1, 2, 3):
        origin = (my - off) % N_DEV
        recv = pltpu.make_async_remote_copy(
            src_ref=out_ref.at[pl.ds(origin * M, M), :],
            dst_ref=out_ref.at[pl.ds(origin * M, M), :],
            send_sem=send_sems.at[off - 1],
            recv_sem=recv_sems.at[off - 1],
            device_id=(origin,),
            device_id_type=pl.DeviceIdType.MESH,
        )
        recv.wait_recv()

    for rdma in sends:
        rdma.wait_send()


def _allgather(yg):
    m, d = yg.shape
    return pl.pallas_call(
        _allgather_body,
        out_shape=jax.ShapeDtypeStruct((N_DEV * m, d), yg.dtype),
        in_specs=[pl.BlockSpec(memory_space=pltpu.VMEM)],
        out_specs=pl.BlockSpec(memory_space=pltpu.VMEM),
        scratch_shapes=[
            pltpu.SemaphoreType.DMA((3,)),
            pltpu.SemaphoreType.DMA((3,)),
        ],
        compiler_params=pltpu.CompilerParams(collective_id=0),
    )(yg)


def kernel(x, router_W, route_idx, expert_W):
    del router_W
    n, d = x.shape
    r = route_idx[:, 0]

    onehot = (r[:, None] == jnp.arange(E, dtype=r.dtype)[None, :])
    pos_cum = jnp.cumsum(onehot.astype(jnp.int32), axis=0)
    mypos = jnp.take_along_axis(pos_cum, r[:, None].astype(jnp.int32), axis=1)[:, 0] - 1
    keep = mypos < CAP
    counts = jnp.sum(onehot.astype(jnp.int32), axis=0)
    starts = jnp.concatenate(
        [jnp.zeros((1,), jnp.int32), jnp.cumsum(counts)[:-1].astype(jnp.int32)]
    )
    order = jnp.argsort(r.astype(jnp.int32) * n + jnp.arange(n, dtype=jnp.int32))
    slot_idx = starts[:, None] + jnp.arange(SLOTS, dtype=jnp.int32)[None, :]
    dispatch = order[jnp.clip(slot_idx, 0, n - 1)]

    i_dev = lax.axis_index("i")
    disp_local = lax.dynamic_slice(dispatch, (i_dev * E_LOCAL, 0), (E_LOCAL, SLOTS))
    xb = x.astype(jnp.bfloat16)
    xg = xb[disp_local.reshape(-1)].reshape(E_LOCAL, SLOTS, d)
    wb = expert_W.astype(jnp.bfloat16)
    yg = jnp.einsum(
        "esd,edh->esh", xg, wb, preferred_element_type=jnp.bfloat16
    ).reshape(M, d)

    table = _allgather(yg)

    flat = r.astype(jnp.int32) * SLOTS + jnp.clip(mypos, 0, SLOTS - 1)
    out = jnp.where(keep[:, None], table[flat], jnp.bfloat16(0))
    return out.astype(jnp.float32)


# device time: 75495 ns/iter; 1.4601x vs baseline; 1.4601x over previous
import jax
import jax.numpy as jnp
from jax import lax
from jax.experimental import pallas as pl
from jax.experimental.pallas import tpu as pltpu

N_DEV = 4
E = 32
E_LOCAL = 8
CAP = 51
SLOTS = 64
M = E_LOCAL * SLOTS


def _body(r_ref, x_ref, w_hbm, out_ref, table, wvm, wsems, send_sems, recv_sems):
    n, d = x_ref.shape
    my = lax.axis_index("i")

    pltpu.make_async_copy(w_hbm.at[0], wvm.at[0], wsems.at[0]).start()

    barrier_sem = pltpu.get_barrier_semaphore()
    for off in (1, 2, 3):
        pl.semaphore_signal(
            barrier_sem, inc=1,
            device_id=((my + off) % N_DEV,),
            device_id_type=pl.DeviceIdType.MESH,
        )
    pl.semaphore_wait(barrier_sem, 3)

    r = r_ref[:, :]
    eh = (r == lax.broadcasted_iota(jnp.int32, (n, E), 1)).astype(jnp.bfloat16)
    tril = (
        lax.broadcasted_iota(jnp.int32, (n, n), 0)
        >= lax.broadcasted_iota(jnp.int32, (n, n), 1)
    ).astype(jnp.bfloat16)
    pos = lax.dot_general(
        tril, eh, (((1,), (0,)), ((), ())), preferred_element_type=jnp.float32
    )
    mypos = (
        jnp.sum(pos * eh.astype(jnp.float32), axis=1, keepdims=True) - 1.0
    ).astype(jnp.int32)
    keep = mypos < CAP

    slotloc = (r - E_LOCAL * my) * SLOTS + mypos
    validd = keep & (r >= E_LOCAL * my) & (r < E_LOCAL * (my + 1))
    disp = (
        (slotloc == lax.broadcasted_iota(jnp.int32, (n, M), 1)) & validd
    ).astype(jnp.bfloat16)
    xb = x_ref[:, :].astype(jnp.bfloat16)
    xg = lax.dot_general(
        disp, xb, (((0,), (0,)), ((), ())), preferred_element_type=jnp.float32
    ).astype(jnp.bfloat16)

    for l in range(E_LOCAL):
        slot = l % 2
        pltpu.make_async_copy(w_hbm.at[l], wvm.at[slot], wsems.at[slot]).wait()
        if l + 1 < E_LOCAL:
            nxt = (l + 1) % 2
            pltpu.make_async_copy(w_hbm.at[l + 1], wvm.at[nxt], wsems.at[nxt]).start()
        wb = wvm[slot, :, :].astype(jnp.bfloat16)
        yl = jnp.dot(
            xg[l * SLOTS:(l + 1) * SLOTS, :], wb,
            preferred_element_type=jnp.float32,
        )
        table[pl.ds(my * M + l * SLOTS, SLOTS), :] = yl.astype(jnp.bfloat16)

    sends = []
    for off in (1, 2, 3):
        rdma = pltpu.make_async_remote_copy(
            src_ref=table.at[pl.ds(my * M, M), :],
            dst_ref=table.at[pl.ds(my * M, M), :],
            send_sem=send_sems.at[off - 1],
            recv_sem=recv_sems.at[off - 1],
            device_id=((my + off) % N_DEV,),
            device_id_type=pl.DeviceIdType.MESH,
        )
        rdma.start()
        sends.append(rdma)

    flat = r * SLOTS + mypos
    comb = (
        (flat == lax.broadcasted_iota(jnp.int32, (n, n), 1)) & keep
    ).astype(jnp.bfloat16)

    for off in (1, 2, 3):
        origin = (my - off) % N_DEV
        recv = pltpu.make_async_remote_copy(
            src_ref=table.at[pl.ds(origin * M, M), :],
            dst_ref=table.at[pl.ds(origin * M, M), :],
            send_sem=send_sems.at[off - 1],
            recv_sem=recv_sems.at[off - 1],
            device_id=(origin,),
            device_id_type=pl.DeviceIdType.MESH,
        )
        recv.wait_recv()

    out_ref[:, :] = lax.dot_general(
        comb, table[:, :], (((1,), (0,)), ((), ())),
        preferred_element_type=jnp.float32,
    )

    for rdma in sends:
        rdma.wait_send()


def kernel(x, router_W, route_idx, expert_W):
    del router_W
    n, d = x.shape
    return pl.pallas_call(
        _body,
        out_shape=jax.ShapeDtypeStruct((n, d), jnp.float32),
        in_specs=[
            pl.BlockSpec(memory_space=pltpu.VMEM),
            pl.BlockSpec(memory_space=pltpu.VMEM),
            pl.BlockSpec(memory_space=pl.ANY),
        ],
        out_specs=pl.BlockSpec(memory_space=pltpu.VMEM),
        scratch_shapes=[
            pltpu.VMEM((E * SLOTS, d), jnp.bfloat16),
            pltpu.VMEM((2, d, d), jnp.float32),
            pltpu.SemaphoreType.DMA((2,)),
            pltpu.SemaphoreType.DMA((3,)),
            pltpu.SemaphoreType.DMA((3,)),
        ],
        compiler_params=pltpu.CompilerParams(collective_id=0),
    )(route_idx, x, expert_W)


# device time: 64927 ns/iter; 1.6977x vs baseline; 1.1628x over previous
import jax
import jax.numpy as jnp
from jax import lax
from jax.experimental import pallas as pl
from jax.experimental.pallas import tpu as pltpu

N_DEV = 4
E = 32
E_LOCAL = 8
CAP = 51
SLOTS = 64
M = E_LOCAL * SLOTS
HALF = M // 2
W_SLOTS = 2


def _body(r_ref, x_ref, w_hbm, out_ref, table, wvm, wsems, send_sems, recv_sems):
    n, d = x_ref.shape
    my = lax.axis_index("i")

    for l in range(W_SLOTS):
        pltpu.make_async_copy(w_hbm.at[l], wvm.at[l], wsems.at[l]).start()

    barrier_sem = pltpu.get_barrier_semaphore()
    for off in (1, 2, 3):
        pl.semaphore_signal(
            barrier_sem, inc=1,
            device_id=((my + off) % N_DEV,),
            device_id_type=pl.DeviceIdType.MESH,
        )
    pl.semaphore_wait(barrier_sem, 3)

    r = r_ref[:, :]
    eh = (r == lax.broadcasted_iota(jnp.int32, (n, E), 1)).astype(jnp.bfloat16)
    tril = (
        lax.broadcasted_iota(jnp.int32, (n, n), 0)
        >= lax.broadcasted_iota(jnp.int32, (n, n), 1)
    ).astype(jnp.bfloat16)
    pos = lax.dot_general(
        tril, eh, (((1,), (0,)), ((), ())), preferred_element_type=jnp.float32
    )
    mypos = (
        jnp.sum(pos * eh.astype(jnp.float32), axis=1, keepdims=True) - 1.0
    ).astype(jnp.int32)
    keep = mypos < CAP

    slotloc = (r - E_LOCAL * my) * SLOTS + mypos
    validd = keep & (r >= E_LOCAL * my) & (r < E_LOCAL * (my + 1))
    disp = (
        (slotloc == lax.broadcasted_iota(jnp.int32, (n, M), 1)) & validd
    ).astype(jnp.bfloat16)
    xb = x_ref[:, :].astype(jnp.bfloat16)
    xg = lax.dot_general(
        disp, xb, (((0,), (0,)), ((), ())), preferred_element_type=jnp.float32
    ).astype(jnp.bfloat16)

    sends = []

    def send_half(h):
        for off in (1, 3, 2):
            rdma = pltpu.make_async_remote_copy(
                src_ref=table.at[pl.ds(my * M + h * HALF, HALF), :],
                dst_ref=table.at[pl.ds(my * M + h * HALF, HALF), :],
                send_sem=send_sems.at[off - 1, h],
                recv_sem=recv_sems.at[off - 1, h],
                device_id=((my + off) % N_DEV,),
                device_id_type=pl.DeviceIdType.MESH,
            )
            rdma.start()
            sends.append(rdma)

    for l in range(E_LOCAL):
        s = l % W_SLOTS
        pltpu.make_async_copy(w_hbm.at[l], wvm.at[s], wsems.at[s]).wait()
        wb = wvm[s, :, :].astype(jnp.bfloat16)
        yl = jnp.dot(
            xg[l * SLOTS:(l + 1) * SLOTS, :], wb,
            preferred_element_type=jnp.float32,
        )
        table[pl.ds(my * M + l * SLOTS, SLOTS), :] = yl.astype(jnp.bfloat16)
        if l + W_SLOTS < E_LOCAL:
            pltpu.make_async_copy(
                w_hbm.at[l + W_SLOTS], wvm.at[s], wsems.at[s]
            ).start()
        if l == E_LOCAL // 2 - 1:
            send_half(0)
    send_half(1)

    flat = r * SLOTS + mypos

    def comb_block(o):
        return (
            ((flat - o * M) == lax.broadcasted_iota(jnp.int32, (n, M), 1)) & keep
        ).astype(jnp.bfloat16)

    acc = lax.dot_general(
        comb_block(my), table[pl.ds(my * M, M), :],
        (((1,), (0,)), ((), ())), preferred_element_type=jnp.float32,
    )
    for off in (1, 3, 2):
        origin = (my - off) % N_DEV
        for h in (0, 1):
            recv = pltpu.make_async_remote_copy(
                src_ref=table.at[pl.ds(origin * M + h * HALF, HALF), :],
                dst_ref=table.at[pl.ds(origin * M + h * HALF, HALF), :],
                send_sem=send_sems.at[off - 1, h],
                recv_sem=recv_sems.at[off - 1, h],
                device_id=(origin,),
                device_id_type=pl.DeviceIdType.MESH,
            )
            recv.wait_recv()
        acc = acc + lax.dot_general(
            comb_block(origin), table[pl.ds(origin * M, M), :],
            (((1,), (0,)), ((), ())), preferred_element_type=jnp.float32,
        )
    out_ref[:, :] = acc

    for rdma in sends:
        rdma.wait_send()


def kernel(x, router_W, route_idx, expert_W):
    del router_W
    n, d = x.shape
    return pl.pallas_call(
        _body,
        out_shape=jax.ShapeDtypeStruct((n, d), jnp.float32),
        in_specs=[
            pl.BlockSpec(memory_space=pltpu.VMEM),
            pl.BlockSpec(memory_space=pltpu.VMEM),
            pl.BlockSpec(memory_space=pl.ANY),
        ],
        out_specs=pl.BlockSpec(memory_space=pltpu.VMEM),
        scratch_shapes=[
            pltpu.VMEM((E * SLOTS, d), jnp.bfloat16),
            pltpu.VMEM((W_SLOTS, d, d), jnp.float32),
            pltpu.SemaphoreType.DMA((W_SLOTS,)),
            pltpu.SemaphoreType.DMA((3, 2)),
            pltpu.SemaphoreType.DMA((3, 2)),
        ],
        compiler_params=pltpu.CompilerParams(
            collective_id=0,
            vmem_limit_bytes=60 * 1024 * 1024,
        ),
    )(route_idx, x, expert_W)
